# baseline (speedup 1.0000x reference)
import functools
import math

import jax
import jax.numpy as jnp
from jax import lax
from jax.experimental import pallas as pl
from jax.experimental.pallas import tpu as pltpu
from jax.experimental.pallas import tpu_sc as plsc

RES = 0.1
TABLE = 1 << 20
D = 120
DP = 128
LA = 120
T = 201
H = 8
DH = D // H
P0, P1, P2 = 73856093, 19349669, 83492791

NC, NS = 2, 16
NW = NC * NS
C = 128
GA = 8
BH = 4096
BA = 2048
PART_UNIT = NW * GA * C
PARTS = 8
UNIT = PARTS * PART_UNIT


def _hash_body(qx_ref, qy_ref, qz_ref, t_ref, h_ref, tm_ref):
    res = jnp.float32(RES)
    gx = jnp.floor(qx_ref[...] / res).astype(jnp.int32)
    gy = jnp.floor(qy_ref[...] / res).astype(jnp.int32)
    gz = jnp.floor(qz_ref[...] / res).astype(jnp.int32)
    s = gx * P0 + gy * P1 + gz * P2
    h_ref[...] = jnp.bitwise_and(s, TABLE - 1)
    tm_ref[...] = jnp.remainder(t_ref[...], T)


def _hash_call(qx2, qy2, qz2, t2d, nrow):
    rb = BH // 128
    grid = nrow // rb
    spec = pl.BlockSpec((rb, 128), lambda i: (i, 0))
    return pl.pallas_call(
        _hash_body,
        grid=(grid,),
        in_specs=[spec, spec, spec, spec],
        out_specs=[spec, spec],
        out_shape=(jax.ShapeDtypeStruct((nrow, 128), jnp.int32),
                   jax.ShapeDtypeStruct((nrow, 128), jnp.int32)),
    )(qx2, qy2, qz2, t2d)


def _pad_body(st_ref, dy_ref, stp_ref, dyp_ref):
    z = jnp.zeros((st_ref.shape[0], DP - D), jnp.float32)
    stp_ref[:, :D] = st_ref[...]
    stp_ref[:, D:] = z
    dyp_ref[:, :D] = dy_ref[...]
    dyp_ref[:, D:] = z


def _pad_call(st, dy):
    v = st.shape[0]
    rb = 1024
    grid = (v + rb - 1) // rb
    ispec = pl.BlockSpec((rb, D), lambda i: (i, 0))
    ospec = pl.BlockSpec((rb, DP), lambda i: (i, 0))
    return pl.pallas_call(
        _pad_body,
        grid=(grid,),
        in_specs=[ispec, ispec],
        out_specs=[ospec, ospec],
        out_shape=(jax.ShapeDtypeStruct((v, DP), jnp.float32),
                   jax.ShapeDtypeStruct((v, DP), jnp.float32)),
    )(st, dy)


def _aux_rows(vvr, tlr, svr, auxr):
    for i in range(GA):
        for j in range(128 // 16):
            sl = pl.ds(j * 16, 16)
            v = vvr[i, sl]
            t = tlr[i, sl]
            svr[i, sl] = jnp.maximum(v, 0)
            auxr[i, sl] = jnp.where(v >= 0, t, -1)


def _lookup_body(ngroup, h_hbm, buf_hbm, t_hbm, sv_hbm, aux_hbm,
                 h0, h1, t0, t1, vv0, vv1, sv0, sv1, ax0, ax1, sem0, sem1):
    cid = lax.axis_index("c")
    sid = lax.axis_index("s")
    wid = sid * NC + cid

    def pair(p, carry):
        r0 = (wid * ngroup + 2 * p) * GA
        r1 = r0 + GA
        pltpu.sync_copy(h_hbm.at[pl.ds(r0, GA)], h0)
        cps0 = [pltpu.async_copy(buf_hbm.at[h0.at[i]], vv0.at[i], sem0)
                for i in range(GA)]
        pltpu.sync_copy(h_hbm.at[pl.ds(r1, GA)], h1)
        cps1 = [pltpu.async_copy(buf_hbm.at[h1.at[i]], vv1.at[i], sem1)
                for i in range(GA)]
        pltpu.sync_copy(t_hbm.at[pl.ds(r0, GA)], t0)
        pltpu.sync_copy(t_hbm.at[pl.ds(r1, GA)], t1)
        for cp in cps0:
            cp.wait()
        _aux_rows(vv0, t0, sv0, ax0)
        pltpu.sync_copy(sv0, sv_hbm.at[pl.ds(r0, GA)])
        pltpu.sync_copy(ax0, aux_hbm.at[pl.ds(r0, GA)])
        for cp in cps1:
            cp.wait()
        _aux_rows(vv1, t1, sv1, ax1)
        pltpu.sync_copy(sv1, sv_hbm.at[pl.ds(r1, GA)])
        pltpu.sync_copy(ax1, aux_hbm.at[pl.ds(r1, GA)])
        return carry

    lax.fori_loop(0, ngroup // 2, pair, 0)


def _lookup_call(h2d, buf, t2d, nrow):
    ngroup = nrow // (NW * GA)
    mesh = plsc.VectorSubcoreMesh(core_axis_name="c", subcore_axis_name="s")
    i2d = lambda: pltpu.VMEM((GA, 128), jnp.int32)
    return pl.kernel(
        functools.partial(_lookup_body, ngroup),
        out_type=(
            jax.ShapeDtypeStruct((nrow, 128), jnp.int32),
            jax.ShapeDtypeStruct((nrow, 128), jnp.int32),
        ),
        mesh=mesh,
        compiler_params=pltpu.CompilerParams(use_tc_tiling_on_sc=True,
                                             needs_layout_passes=False),
        scratch_types=[i2d(), i2d(), i2d(), i2d(), i2d(), i2d(),
                       i2d(), i2d(), i2d(), i2d(),
                       pltpu.SemaphoreType.DMA, pltpu.SemaphoreType.DMA],
    )(h2d, buf, t2d)


def _embed_aux(axb, i, strow):
    cidx = jnp.full((16,), LA, jnp.int32)
    for j in range(C // 16):
        ridx = lax.broadcasted_iota(jnp.int32, (16,), 0) + j * 16
        af = axb[i, pl.ds(j * 16, 16)].astype(jnp.float32)
        plsc.store_scatter(strow, [ridx, cidx], af)


def _rows_body(nchunk, sv_hbm, aux_hbm, st_hbm, dy_hbm, stg_hbm, dyg_hbm,
               svb, axb, str0, str1, str2, dyr0, dyr1, dyr2,
               gs0, gs1, gs2, gd0, gd1, gd2, sem_w):
    cid = lax.axis_index("c")
    sid = lax.axis_index("s")
    wid = sid * NC + cid
    strs = (str0, str1, str2)
    dyrs = (dyr0, dyr1, dyr2)
    gss = (gs0, gs1, gs2)
    gds = (gd0, gd1, gd2)

    def octet(o, carry):
        base = wid * nchunk + o * 8
        pltpu.sync_copy(sv_hbm.at[pl.ds(base, 8)], svb)
        pltpu.sync_copy(aux_hbm.at[pl.ds(base, 8)], axb)

        def fire(i):
            k = i % 3
            return (pltpu.async_copy(st_hbm.at[svb.at[i]], strs[k], gss[k]),
                    pltpu.async_copy(dy_hbm.at[svb.at[i]], dyrs[k], gds[k]))

        cps = {i: fire(i) for i in range(3)}
        for i in range(8):
            k = i % 3
            cs, cd = cps[i]
            cs.wait()
            cd.wait()
            _embed_aux(axb, i, strs[k])
            ws = pltpu.async_copy(strs[k], stg_hbm.at[pl.ds((base + i) * C, C)],
                                  sem_w)
            wd = pltpu.async_copy(dyrs[k], dyg_hbm.at[pl.ds((base + i) * C, C)],
                                  sem_w)
            ws.wait()
            wd.wait()
            if i + 3 < 8:
                cps[i + 3] = fire(i + 3)
        return carry

    lax.fori_loop(0, nchunk // 8, octet, 0)


def _rows_call(sv2d, aux2d, stp, dyp, mp):
    nchunk = mp // (NW * C)
    mesh = plsc.VectorSubcoreMesh(core_axis_name="c", subcore_axis_name="s")
    rows = lambda: pltpu.VMEM((C, DP), jnp.float32)
    i2d = lambda: pltpu.VMEM((8, 128), jnp.int32)
    sem = pltpu.SemaphoreType.DMA
    return pl.kernel(
        functools.partial(_rows_body, nchunk),
        out_type=(
            jax.ShapeDtypeStruct((mp, DP), jnp.float32),
            jax.ShapeDtypeStruct((mp, DP), jnp.float32),
        ),
        mesh=mesh,
        compiler_params=pltpu.CompilerParams(use_tc_tiling_on_sc=True,
                                             needs_layout_passes=False),
        scratch_types=[i2d(), i2d(),
                       rows(), rows(), rows(), rows(), rows(), rows(),
                       sem, sem, sem, sem, sem, sem, sem],
    )(sv2d, aux2d, stp, dyp)


def _fuse(a16, b16, wqv, wkv, bq, bv, bds, wo, bo):
    f32 = jnp.float32
    qv = jnp.dot(a16, wqv, preferred_element_type=f32)
    qa = qv[:, :DP] + bq
    va = qv[:, DP:] + bv
    cat = jnp.concatenate([a16, b16], axis=1)
    kv = jnp.dot(cat, wkv, preferred_element_type=f32)
    dk = kv[:, :DP]
    dv = kv[:, DP:]
    pd = jnp.dot((qa * dk).astype(jnp.bfloat16), bds,
                 preferred_element_type=f32)
    w1 = 1.0 / (1.0 + jnp.exp(-pd))
    o = va + w1 * dv
    o16 = o.astype(jnp.bfloat16)
    r = jnp.dot(o16, wo, preferred_element_type=f32)
    if bo.dtype == jnp.bfloat16:
        return (r + bo.astype(f32)).astype(jnp.bfloat16)
    return r + bo


def _attn_body(stg_ref, dyg_ref, te_ref,
               wqv1, wkv1, bq1, bv1, wo1, bo1,
               wqv2, wkv2, bq2, bv2, wo2, bo2,
               bds_ref, *rest):
    out_ref = rest[-1]
    stb = stg_ref[...]
    aux = stb[:, LA:LA + 1]
    ti = aux.astype(jnp.int32)
    oh = (ti == lax.broadcasted_iota(jnp.int32, (BA, T), 1))
    te16 = jnp.dot(oh.astype(jnp.bfloat16), te_ref[...],
                   preferred_element_type=jnp.float32
                   ).astype(jnp.bfloat16)
    a1 = dyg_ref[...].astype(jnp.bfloat16)
    a2 = stb.astype(jnp.bfloat16)
    bds = bds_ref[...]
    cond16 = _fuse(a1, te16, wqv1[...], wkv1[...], bq1[...], bv1[...],
                   bds, wo1[...], bo1[...])
    fused = _fuse(a2, cond16, wqv2[...], wkv2[...], bq2[...], bv2[...],
                  bds, wo2[...], bo2[...])
    out_ref[...] = jnp.where(aux >= 0.0, fused, jnp.float32(0.0))


def _attn_call(nblk, stg, dyg, te16, w1, w2, bds16):
    full = lambda shape: pl.BlockSpec(shape, lambda i: tuple(0 for _ in shape))
    wspecs = lambda w: [full(x.shape) for x in w]
    specs = ([
        pl.BlockSpec((BA, DP), lambda i: (i, 0)),
        pl.BlockSpec((BA, DP), lambda i: (i, 0)),
        full((T, DP)),
    ] + wspecs(w1) + wspecs(w2) + [full((DP, DP))])
    args = [stg, dyg, te16] + list(w1) + list(w2) + [bds16]
    return pl.pallas_call(
        _attn_body,
        grid=(nblk,),
        in_specs=specs,
        out_specs=pl.BlockSpec((BA, D), lambda i: (i, 0)),
        out_shape=jax.ShapeDtypeStruct((nblk * BA, D), jnp.float32),
    )(*args)


def kernel(query_pts, query_times, buffer_voxel_index, static_features,
           dynamic_features, time_embeddings,
           f1_Wqkv, f1_bqkv, f1_Wo, f1_bo,
           f2_Wqkv, f2_bqkv, f2_Wo, f2_bo):
    m = query_pts.shape[0]
    mp = ((m + UNIT - 1) // UNIT) * UNIT
    nrow = mp // 128

    pts = query_pts.astype(jnp.float32)
    pidx = jnp.arange(mp - m, dtype=jnp.int32)
    padx = (pidx % 10).astype(jnp.float32) * 0.1 + 0.05
    pady = ((pidx // 10) % 10).astype(jnp.float32) * 0.1 + 0.05
    padz = ((pidx // 100) % 10).astype(jnp.float32) * 0.1 + 0.05
    qx2 = jnp.concatenate([pts[:, 0], padx]).reshape(nrow, 128)
    qy2 = jnp.concatenate([pts[:, 1], pady]).reshape(nrow, 128)
    qz2 = jnp.concatenate([pts[:, 2], padz]).reshape(nrow, 128)
    t2d = jnp.pad(query_times.astype(jnp.int32),
                  (0, mp - m)).reshape(nrow, 128)
    buf = buffer_voxel_index.astype(jnp.int32)
    stp = jnp.pad(static_features.astype(jnp.float32),
                  ((0, 0), (0, DP - D)))
    dyp = jnp.pad(dynamic_features.astype(jnp.float32),
                  ((0, 0), (0, DP - D)))

    h2d, tm2d = _hash_call(qx2, qy2, qz2, t2d, nrow)
    sv2d, aux2d = _lookup_call(h2d, buf, tm2d, nrow)

    ps = mp // PARTS
    pr = nrow // PARTS
    gathered = [_rows_call(sv2d[p * pr:(p + 1) * pr],
                           aux2d[p * pr:(p + 1) * pr], stp, dyp, ps)
                for p in range(PARTS)]

    bf16 = jnp.bfloat16

    def pad2(w, rows, cols):
        return jnp.pad(w, ((0, rows - w.shape[0]), (0, cols - w.shape[1])))

    def wpack(wqkv, bqkv, wo, bo, last):
        wq = pad2(wqkv[:, :D], DP, DP)
        wk = pad2(wqkv[:, D:2 * D], DP, DP)
        wv = pad2(wqkv[:, 2 * D:], DP, DP)
        wqv = jnp.concatenate([wq, wv], axis=1).astype(bf16)
        top = jnp.concatenate([-wk, -wv], axis=1)
        bot = jnp.concatenate([wk, wv], axis=1)
        wkv = jnp.concatenate([top, bot], axis=0).astype(bf16)
        bq = jnp.pad(bqkv[:D], (0, DP - D)).reshape(1, DP)
        bv = jnp.pad(bqkv[2 * D:], (0, DP - D)).reshape(1, DP)
        if last:
            wop = jnp.pad(wo, ((0, DP - D), (0, 0))).astype(bf16)
            bop = bo.reshape(1, D).astype(jnp.float32)
        else:
            wop = pad2(wo, DP, DP).astype(bf16)
            bop = jnp.pad(bo, (0, DP - D)).reshape(1, DP).astype(bf16)
        return (wqv, wkv, bq.astype(jnp.float32), bv.astype(jnp.float32),
                wop, bop)

    w1 = wpack(f1_Wqkv, f1_bqkv, f1_Wo, f1_bo, last=False)
    w2 = wpack(f2_Wqkv, f2_bqkv, f2_Wo, f2_bo, last=True)

    ri = jnp.arange(DP) // DH
    bd = jnp.where((ri[:, None] == ri[None, :])
                   & (jnp.arange(DP)[:, None] < D)
                   & (jnp.arange(DP)[None, :] < D),
                   1.0 / math.sqrt(DH), 0.0)
    bds16 = bd.astype(bf16)
    te16 = jnp.pad(time_embeddings, ((0, 0), (0, DP - D))).astype(bf16)

    nb = (m + BA - 1) // BA
    pb = ps // BA
    outs = []
    for p in range(PARTS):
        nbp = min(pb, nb - p * pb)
        if nbp <= 0:
            break
        stg_p, dyg_p = gathered[p]
        outs.append(_attn_call(nbp, stg_p, dyg_p, te16, w1, w2, bds16))
    if len(outs) == 1:
        return outs[0][:m]
    return jnp.concatenate(outs, axis=0)[:m]


# speedup vs baseline: 1.0578x; 1.0578x over previous
import functools
import math

import jax
import jax.numpy as jnp
from jax import lax
from jax.experimental import pallas as pl
from jax.experimental.pallas import tpu as pltpu
from jax.experimental.pallas import tpu_sc as plsc

RES = 0.1
TABLE = 1 << 20
D = 120
DP = 128
LA = 120
T = 201
H = 8
DH = D // H
P0, P1, P2 = 73856093, 19349669, 83492791

NC, NS = 2, 16
NW = NC * NS
C = 128
GA = 8
BH = 4096
BA = 2048
PART_UNIT = NW * GA * C
PARTS = 4
UNIT = PARTS * PART_UNIT


def _hash_body(qx_ref, qy_ref, qz_ref, t_ref, h_ref, tm_ref):
    res = jnp.float32(RES)
    gx = jnp.floor(qx_ref[...] / res).astype(jnp.int32)
    gy = jnp.floor(qy_ref[...] / res).astype(jnp.int32)
    gz = jnp.floor(qz_ref[...] / res).astype(jnp.int32)
    s = gx * P0 + gy * P1 + gz * P2
    h_ref[...] = jnp.bitwise_and(s, TABLE - 1)
    tm_ref[...] = jnp.remainder(t_ref[...], T)


def _hash_call(qx2, qy2, qz2, t2d, nrow):
    rb = BH // 128
    grid = nrow // rb
    spec = pl.BlockSpec((rb, 128), lambda i: (i, 0))
    return pl.pallas_call(
        _hash_body,
        grid=(grid,),
        in_specs=[spec, spec, spec, spec],
        out_specs=[spec, spec],
        out_shape=(jax.ShapeDtypeStruct((nrow, 128), jnp.int32),
                   jax.ShapeDtypeStruct((nrow, 128), jnp.int32)),
    )(qx2, qy2, qz2, t2d)


def _pad_body(st_ref, dy_ref, stp_ref, dyp_ref):
    z = jnp.zeros((st_ref.shape[0], DP - D), jnp.float32)
    stp_ref[:, :D] = st_ref[...]
    stp_ref[:, D:] = z
    dyp_ref[:, :D] = dy_ref[...]
    dyp_ref[:, D:] = z


def _pad_call(st, dy):
    v = st.shape[0]
    rb = 1024
    grid = (v + rb - 1) // rb
    ispec = pl.BlockSpec((rb, D), lambda i: (i, 0))
    ospec = pl.BlockSpec((rb, DP), lambda i: (i, 0))
    return pl.pallas_call(
        _pad_body,
        grid=(grid,),
        in_specs=[ispec, ispec],
        out_specs=[ospec, ospec],
        out_shape=(jax.ShapeDtypeStruct((v, DP), jnp.float32),
                   jax.ShapeDtypeStruct((v, DP), jnp.float32)),
    )(st, dy)


def _aux_rows(vvr, tlr, svr, auxr):
    for i in range(GA):
        for j in range(128 // 16):
            sl = pl.ds(j * 16, 16)
            v = vvr[i, sl]
            t = tlr[i, sl]
            svr[i, sl] = jnp.maximum(v, 0)
            auxr[i, sl] = jnp.where(v >= 0, t, -1)


def _lookup_body(ngroup, h_hbm, buf_hbm, t_hbm, sv_hbm, aux_hbm,
                 h0, h1, t0, t1, vv0, vv1, sv0, sv1, ax0, ax1, sem0, sem1):
    cid = lax.axis_index("c")
    sid = lax.axis_index("s")
    wid = sid * NC + cid

    def pair(p, carry):
        r0 = (wid * ngroup + 2 * p) * GA
        r1 = r0 + GA
        pltpu.sync_copy(h_hbm.at[pl.ds(r0, GA)], h0)
        cps0 = [pltpu.async_copy(buf_hbm.at[h0.at[i]], vv0.at[i], sem0)
                for i in range(GA)]
        pltpu.sync_copy(h_hbm.at[pl.ds(r1, GA)], h1)
        cps1 = [pltpu.async_copy(buf_hbm.at[h1.at[i]], vv1.at[i], sem1)
                for i in range(GA)]
        pltpu.sync_copy(t_hbm.at[pl.ds(r0, GA)], t0)
        pltpu.sync_copy(t_hbm.at[pl.ds(r1, GA)], t1)
        for cp in cps0:
            cp.wait()
        _aux_rows(vv0, t0, sv0, ax0)
        pltpu.sync_copy(sv0, sv_hbm.at[pl.ds(r0, GA)])
        pltpu.sync_copy(ax0, aux_hbm.at[pl.ds(r0, GA)])
        for cp in cps1:
            cp.wait()
        _aux_rows(vv1, t1, sv1, ax1)
        pltpu.sync_copy(sv1, sv_hbm.at[pl.ds(r1, GA)])
        pltpu.sync_copy(ax1, aux_hbm.at[pl.ds(r1, GA)])
        return carry

    lax.fori_loop(0, ngroup // 2, pair, 0)


def _lookup_call(h2d, buf, t2d, nrow):
    ngroup = nrow // (NW * GA)
    mesh = plsc.VectorSubcoreMesh(core_axis_name="c", subcore_axis_name="s")
    i2d = lambda: pltpu.VMEM((GA, 128), jnp.int32)
    return pl.kernel(
        functools.partial(_lookup_body, ngroup),
        out_type=(
            jax.ShapeDtypeStruct((nrow, 128), jnp.int32),
            jax.ShapeDtypeStruct((nrow, 128), jnp.int32),
        ),
        mesh=mesh,
        compiler_params=pltpu.CompilerParams(use_tc_tiling_on_sc=True,
                                             needs_layout_passes=False),
        scratch_types=[i2d(), i2d(), i2d(), i2d(), i2d(), i2d(),
                       i2d(), i2d(), i2d(), i2d(),
                       pltpu.SemaphoreType.DMA, pltpu.SemaphoreType.DMA],
    )(h2d, buf, t2d)


def _embed_aux(axb, i, strow):
    cidx = jnp.full((16,), LA, jnp.int32)
    for j in range(C // 16):
        ridx = lax.broadcasted_iota(jnp.int32, (16,), 0) + j * 16
        af = axb[i, pl.ds(j * 16, 16)].astype(jnp.float32)
        plsc.store_scatter(strow, [ridx, cidx], af)


def _rows_body(nchunk, sv_hbm, aux_hbm, st_hbm, dy_hbm, stg_hbm, dyg_hbm,
               svb, axb, str0, str1, str2, dyr0, dyr1, dyr2,
               gs0, gs1, gs2, gd0, gd1, gd2, sem_w):
    cid = lax.axis_index("c")
    sid = lax.axis_index("s")
    wid = sid * NC + cid
    strs = (str0, str1, str2)
    dyrs = (dyr0, dyr1, dyr2)
    gss = (gs0, gs1, gs2)
    gds = (gd0, gd1, gd2)

    def octet(o, carry):
        base = wid * nchunk + o * 8
        pltpu.sync_copy(sv_hbm.at[pl.ds(base, 8)], svb)
        pltpu.sync_copy(aux_hbm.at[pl.ds(base, 8)], axb)

        def fire(i):
            k = i % 3
            return (pltpu.async_copy(st_hbm.at[svb.at[i]], strs[k], gss[k]),
                    pltpu.async_copy(dy_hbm.at[svb.at[i]], dyrs[k], gds[k]))

        cps = {i: fire(i) for i in range(3)}
        for i in range(8):
            k = i % 3
            cs, cd = cps[i]
            cs.wait()
            cd.wait()
            _embed_aux(axb, i, strs[k])
            ws = pltpu.async_copy(strs[k], stg_hbm.at[pl.ds((base + i) * C, C)],
                                  sem_w)
            wd = pltpu.async_copy(dyrs[k], dyg_hbm.at[pl.ds((base + i) * C, C)],
                                  sem_w)
            ws.wait()
            wd.wait()
            if i + 3 < 8:
                cps[i + 3] = fire(i + 3)
        return carry

    lax.fori_loop(0, nchunk // 8, octet, 0)


def _rows_call(sv2d, aux2d, stp, dyp, mp):
    nchunk = mp // (NW * C)
    mesh = plsc.VectorSubcoreMesh(core_axis_name="c", subcore_axis_name="s")
    rows = lambda: pltpu.VMEM((C, DP), jnp.float32)
    i2d = lambda: pltpu.VMEM((8, 128), jnp.int32)
    sem = pltpu.SemaphoreType.DMA
    return pl.kernel(
        functools.partial(_rows_body, nchunk),
        out_type=(
            jax.ShapeDtypeStruct((mp, DP), jnp.float32),
            jax.ShapeDtypeStruct((mp, DP), jnp.float32),
        ),
        mesh=mesh,
        compiler_params=pltpu.CompilerParams(use_tc_tiling_on_sc=True,
                                             needs_layout_passes=False),
        scratch_types=[i2d(), i2d(),
                       rows(), rows(), rows(), rows(), rows(), rows(),
                       sem, sem, sem, sem, sem, sem, sem],
    )(sv2d, aux2d, stp, dyp)


def _fuse(a16, b16, wqv, wkv, bq, bv, bds, wo, bo):
    f32 = jnp.float32
    qv = jnp.dot(a16, wqv, preferred_element_type=f32)
    qa = qv[:, :DP] + bq
    va = qv[:, DP:] + bv
    cat = jnp.concatenate([a16, b16], axis=1)
    kv = jnp.dot(cat, wkv, preferred_element_type=f32)
    dk = kv[:, :DP]
    dv = kv[:, DP:]
    pd = jnp.dot((qa * dk).astype(jnp.bfloat16), bds,
                 preferred_element_type=f32)
    w1 = 1.0 / (1.0 + jnp.exp(-pd))
    o = va + w1 * dv
    o16 = o.astype(jnp.bfloat16)
    r = jnp.dot(o16, wo, preferred_element_type=f32)
    if bo.dtype == jnp.bfloat16:
        return (r + bo.astype(f32)).astype(jnp.bfloat16)
    return r + bo


def _attn_body(stg_ref, dyg_ref, te_ref,
               wqv1, wkv1, bq1, bv1, wo1, bo1,
               wqv2, wkv2, bq2, bv2, wo2, bo2,
               bds_ref, *rest):
    out_ref = rest[-1]
    stb = stg_ref[...]
    aux = stb[:, LA:LA + 1]
    ti = aux.astype(jnp.int32)
    oh = (ti == lax.broadcasted_iota(jnp.int32, (BA, T), 1))
    te16 = jnp.dot(oh.astype(jnp.bfloat16), te_ref[...],
                   preferred_element_type=jnp.float32
                   ).astype(jnp.bfloat16)
    a1 = dyg_ref[...].astype(jnp.bfloat16)
    a2 = stb.astype(jnp.bfloat16)
    bds = bds_ref[...]
    cond16 = _fuse(a1, te16, wqv1[...], wkv1[...], bq1[...], bv1[...],
                   bds, wo1[...], bo1[...])
    fused = _fuse(a2, cond16, wqv2[...], wkv2[...], bq2[...], bv2[...],
                  bds, wo2[...], bo2[...])
    out_ref[...] = jnp.where(aux >= 0.0, fused, jnp.float32(0.0))


def _attn_call(nblk, stg, dyg, te16, w1, w2, bds16):
    full = lambda shape: pl.BlockSpec(shape, lambda i: tuple(0 for _ in shape))
    wspecs = lambda w: [full(x.shape) for x in w]
    specs = ([
        pl.BlockSpec((BA, DP), lambda i: (i, 0)),
        pl.BlockSpec((BA, DP), lambda i: (i, 0)),
        full((T, DP)),
    ] + wspecs(w1) + wspecs(w2) + [full((DP, DP))])
    args = [stg, dyg, te16] + list(w1) + list(w2) + [bds16]
    return pl.pallas_call(
        _attn_body,
        grid=(nblk,),
        in_specs=specs,
        out_specs=pl.BlockSpec((BA, D), lambda i: (i, 0)),
        out_shape=jax.ShapeDtypeStruct((nblk * BA, D), jnp.float32),
    )(*args)


def kernel(query_pts, query_times, buffer_voxel_index, static_features,
           dynamic_features, time_embeddings,
           f1_Wqkv, f1_bqkv, f1_Wo, f1_bo,
           f2_Wqkv, f2_bqkv, f2_Wo, f2_bo):
    m = query_pts.shape[0]
    mp = ((m + UNIT - 1) // UNIT) * UNIT
    nrow = mp // 128

    pts = query_pts.astype(jnp.float32)
    pidx = jnp.arange(mp - m, dtype=jnp.int32)
    padx = (pidx % 10).astype(jnp.float32) * 0.1 + 0.05
    pady = ((pidx // 10) % 10).astype(jnp.float32) * 0.1 + 0.05
    padz = ((pidx // 100) % 10).astype(jnp.float32) * 0.1 + 0.05
    qx2 = jnp.concatenate([pts[:, 0], padx]).reshape(nrow, 128)
    qy2 = jnp.concatenate([pts[:, 1], pady]).reshape(nrow, 128)
    qz2 = jnp.concatenate([pts[:, 2], padz]).reshape(nrow, 128)
    t2d = jnp.pad(query_times.astype(jnp.int32),
                  (0, mp - m)).reshape(nrow, 128)
    buf = buffer_voxel_index.astype(jnp.int32)
    stp = jnp.pad(static_features.astype(jnp.float32),
                  ((0, 0), (0, DP - D)))
    dyp = jnp.pad(dynamic_features.astype(jnp.float32),
                  ((0, 0), (0, DP - D)))

    h2d, tm2d = _hash_call(qx2, qy2, qz2, t2d, nrow)
    sv2d, aux2d = _lookup_call(h2d, buf, tm2d, nrow)

    ps = mp // PARTS
    pr = nrow // PARTS
    gathered = [_rows_call(sv2d[p * pr:(p + 1) * pr],
                           aux2d[p * pr:(p + 1) * pr], stp, dyp, ps)
                for p in range(PARTS)]

    bf16 = jnp.bfloat16

    def pad2(w, rows, cols):
        return jnp.pad(w, ((0, rows - w.shape[0]), (0, cols - w.shape[1])))

    def wpack(wqkv, bqkv, wo, bo, last):
        wq = pad2(wqkv[:, :D], DP, DP)
        wk = pad2(wqkv[:, D:2 * D], DP, DP)
        wv = pad2(wqkv[:, 2 * D:], DP, DP)
        wqv = jnp.concatenate([wq, wv], axis=1).astype(bf16)
        top = jnp.concatenate([-wk, -wv], axis=1)
        bot = jnp.concatenate([wk, wv], axis=1)
        wkv = jnp.concatenate([top, bot], axis=0).astype(bf16)
        bq = jnp.pad(bqkv[:D], (0, DP - D)).reshape(1, DP)
        bv = jnp.pad(bqkv[2 * D:], (0, DP - D)).reshape(1, DP)
        if last:
            wop = jnp.pad(wo, ((0, DP - D), (0, 0))).astype(bf16)
            bop = bo.reshape(1, D).astype(jnp.float32)
        else:
            wop = pad2(wo, DP, DP).astype(bf16)
            bop = jnp.pad(bo, (0, DP - D)).reshape(1, DP).astype(bf16)
        return (wqv, wkv, bq.astype(jnp.float32), bv.astype(jnp.float32),
                wop, bop)

    w1 = wpack(f1_Wqkv, f1_bqkv, f1_Wo, f1_bo, last=False)
    w2 = wpack(f2_Wqkv, f2_bqkv, f2_Wo, f2_bo, last=True)

    ri = jnp.arange(DP) // DH
    bd = jnp.where((ri[:, None] == ri[None, :])
                   & (jnp.arange(DP)[:, None] < D)
                   & (jnp.arange(DP)[None, :] < D),
                   1.0 / math.sqrt(DH), 0.0)
    bds16 = bd.astype(bf16)
    te16 = jnp.pad(time_embeddings, ((0, 0), (0, DP - D))).astype(bf16)

    nb = (m + BA - 1) // BA
    pb = ps // BA
    outs = []
    for p in range(PARTS):
        nbp = min(pb, nb - p * pb)
        if nbp <= 0:
            break
        stg_p, dyg_p = gathered[p]
        outs.append(_attn_call(nbp, stg_p, dyg_p, te16, w1, w2, bds16))
    if len(outs) == 1:
        return outs[0][:m]
    return jnp.concatenate(outs, axis=0)[:m]


# speedup vs baseline: 1.0598x; 1.0019x over previous
import functools
import math

import jax
import jax.numpy as jnp
from jax import lax
from jax.experimental import pallas as pl
from jax.experimental.pallas import tpu as pltpu
from jax.experimental.pallas import tpu_sc as plsc

RES = 0.1
TABLE = 1 << 20
D = 120
DP = 128
LA = 120
T = 201
H = 8
DH = D // H
P0, P1, P2 = 73856093, 19349669, 83492791

NC, NS = 2, 16
NW = NC * NS
C = 128
GA = 8
BH = 4096
BA = 2048
PART_UNIT = NW * GA * C
PARTS = 4
UNIT = PARTS * PART_UNIT


def _hash_body(qx_ref, qy_ref, qz_ref, t_ref, h_ref, tm_ref):
    res = jnp.float32(RES)
    gx = jnp.floor(qx_ref[...] / res).astype(jnp.int32)
    gy = jnp.floor(qy_ref[...] / res).astype(jnp.int32)
    gz = jnp.floor(qz_ref[...] / res).astype(jnp.int32)
    s = gx * P0 + gy * P1 + gz * P2
    h_ref[...] = jnp.bitwise_and(s, TABLE - 1)
    tm_ref[...] = jnp.remainder(t_ref[...], T)


def _hash_call(qx2, qy2, qz2, t2d, nrow):
    rb = BH // 128
    grid = nrow // rb
    spec = pl.BlockSpec((rb, 128), lambda i: (i, 0))
    return pl.pallas_call(
        _hash_body,
        grid=(grid,),
        in_specs=[spec, spec, spec, spec],
        out_specs=[spec, spec],
        out_shape=(jax.ShapeDtypeStruct((nrow, 128), jnp.int32),
                   jax.ShapeDtypeStruct((nrow, 128), jnp.int32)),
    )(qx2, qy2, qz2, t2d)


def _pad_body(st_ref, dy_ref, stp_ref, dyp_ref):
    z = jnp.zeros((st_ref.shape[0], DP - D), jnp.float32)
    stp_ref[:, :D] = st_ref[...]
    stp_ref[:, D:] = z
    dyp_ref[:, :D] = dy_ref[...]
    dyp_ref[:, D:] = z


def _pad_call(st, dy):
    v = st.shape[0]
    rb = 1024
    grid = (v + rb - 1) // rb
    ispec = pl.BlockSpec((rb, D), lambda i: (i, 0))
    ospec = pl.BlockSpec((rb, DP), lambda i: (i, 0))
    return pl.pallas_call(
        _pad_body,
        grid=(grid,),
        in_specs=[ispec, ispec],
        out_specs=[ospec, ospec],
        out_shape=(jax.ShapeDtypeStruct((v, DP), jnp.float32),
                   jax.ShapeDtypeStruct((v, DP), jnp.float32)),
    )(st, dy)


def _aux_rows(vvr, tlr, svr, auxr):
    for i in range(GA):
        for j in range(128 // 16):
            sl = pl.ds(j * 16, 16)
            v = vvr[i, sl]
            t = tlr[i, sl]
            svr[i, sl] = jnp.maximum(v, 0)
            auxr[i, sl] = jnp.where(v >= 0, t, -1)


def _lookup_body(ngroup, h_hbm, buf_hbm, t_hbm, sv_hbm, aux_hbm,
                 h0, h1, t0, t1, vv0, vv1, sv0, sv1, ax0, ax1, sem0, sem1):
    cid = lax.axis_index("c")
    sid = lax.axis_index("s")
    wid = sid * NC + cid

    def pair(p, carry):
        r0 = (wid * ngroup + 2 * p) * GA
        r1 = r0 + GA
        pltpu.sync_copy(h_hbm.at[pl.ds(r0, GA)], h0)
        cps0 = [pltpu.async_copy(buf_hbm.at[h0.at[i]], vv0.at[i], sem0)
                for i in range(GA)]
        pltpu.sync_copy(h_hbm.at[pl.ds(r1, GA)], h1)
        cps1 = [pltpu.async_copy(buf_hbm.at[h1.at[i]], vv1.at[i], sem1)
                for i in range(GA)]
        pltpu.sync_copy(t_hbm.at[pl.ds(r0, GA)], t0)
        pltpu.sync_copy(t_hbm.at[pl.ds(r1, GA)], t1)
        for cp in cps0:
            cp.wait()
        _aux_rows(vv0, t0, sv0, ax0)
        pltpu.sync_copy(sv0, sv_hbm.at[pl.ds(r0, GA)])
        pltpu.sync_copy(ax0, aux_hbm.at[pl.ds(r0, GA)])
        for cp in cps1:
            cp.wait()
        _aux_rows(vv1, t1, sv1, ax1)
        pltpu.sync_copy(sv1, sv_hbm.at[pl.ds(r1, GA)])
        pltpu.sync_copy(ax1, aux_hbm.at[pl.ds(r1, GA)])
        return carry

    lax.fori_loop(0, ngroup // 2, pair, 0)


def _lookup_call(h2d, buf, t2d, nrow):
    ngroup = nrow // (NW * GA)
    mesh = plsc.VectorSubcoreMesh(core_axis_name="c", subcore_axis_name="s")
    i2d = lambda: pltpu.VMEM((GA, 128), jnp.int32)
    return pl.kernel(
        functools.partial(_lookup_body, ngroup),
        out_type=(
            jax.ShapeDtypeStruct((nrow, 128), jnp.int32),
            jax.ShapeDtypeStruct((nrow, 128), jnp.int32),
        ),
        mesh=mesh,
        compiler_params=pltpu.CompilerParams(use_tc_tiling_on_sc=True,
                                             needs_layout_passes=False),
        scratch_types=[i2d(), i2d(), i2d(), i2d(), i2d(), i2d(),
                       i2d(), i2d(), i2d(), i2d(),
                       pltpu.SemaphoreType.DMA, pltpu.SemaphoreType.DMA],
    )(h2d, buf, t2d)


def _embed_aux(axb, i, strow):
    cidx = jnp.full((16,), LA, jnp.int32)
    for j in range(C // 16):
        ridx = lax.broadcasted_iota(jnp.int32, (16,), 0) + j * 16
        af = axb[i, pl.ds(j * 16, 16)].astype(jnp.float32)
        plsc.store_scatter(strow, [ridx, cidx], af)


def _rows_body(nchunk, sv_hbm, aux_hbm, st_hbm, dy_hbm, stg_hbm, dyg_hbm,
               svb, axb, str0, str1, str2, dyr0, dyr1, dyr2,
               gs0, gs1, gs2, gd0, gd1, gd2, sem_w):
    cid = lax.axis_index("c")
    sid = lax.axis_index("s")
    wid = sid * NC + cid
    strs = (str0, str1, str2)
    dyrs = (dyr0, dyr1, dyr2)
    gss = (gs0, gs1, gs2)
    gds = (gd0, gd1, gd2)

    def octet(o, carry):
        base = wid * nchunk + o * 8
        pltpu.sync_copy(sv_hbm.at[pl.ds(base, 8)], svb)
        pltpu.sync_copy(aux_hbm.at[pl.ds(base, 8)], axb)

        def fire(i):
            k = i % 3
            return (pltpu.async_copy(st_hbm.at[svb.at[i]], strs[k], gss[k]),
                    pltpu.async_copy(dy_hbm.at[svb.at[i]], dyrs[k], gds[k]))

        cps = {i: fire(i) for i in range(3)}
        for i in range(8):
            k = i % 3
            cs, cd = cps[i]
            cs.wait()
            cd.wait()
            _embed_aux(axb, i, strs[k])
            ws = pltpu.async_copy(strs[k], stg_hbm.at[pl.ds((base + i) * C, C)],
                                  sem_w)
            wd = pltpu.async_copy(dyrs[k], dyg_hbm.at[pl.ds((base + i) * C, C)],
                                  sem_w)
            ws.wait()
            wd.wait()
            if i + 3 < 8:
                cps[i + 3] = fire(i + 3)
        return carry

    lax.fori_loop(0, nchunk // 8, octet, 0)


def _rows_call(sv2d, aux2d, stp, dyp, mp):
    nchunk = mp // (NW * C)
    mesh = plsc.VectorSubcoreMesh(core_axis_name="c", subcore_axis_name="s")
    rows = lambda: pltpu.VMEM((C, DP), jnp.float32)
    i2d = lambda: pltpu.VMEM((8, 128), jnp.int32)
    sem = pltpu.SemaphoreType.DMA
    return pl.kernel(
        functools.partial(_rows_body, nchunk),
        out_type=(
            jax.ShapeDtypeStruct((mp, DP), jnp.float32),
            jax.ShapeDtypeStruct((mp, DP), jnp.float32),
        ),
        mesh=mesh,
        compiler_params=pltpu.CompilerParams(use_tc_tiling_on_sc=True,
                                             needs_layout_passes=False),
        scratch_types=[i2d(), i2d(),
                       rows(), rows(), rows(), rows(), rows(), rows(),
                       sem, sem, sem, sem, sem, sem, sem],
    )(sv2d, aux2d, stp, dyp)


def _fuse(a16, b16, wqv, wkv, bq, bv, bds, wo, bo):
    f32 = jnp.float32
    qv = jnp.dot(a16, wqv, preferred_element_type=f32)
    qa = qv[:, :DP] + bq
    va = qv[:, DP:] + bv
    cat = jnp.concatenate([a16, b16], axis=1)
    kv = jnp.dot(cat, wkv, preferred_element_type=f32)
    dk = kv[:, :DP]
    dv = kv[:, DP:]
    pd = jnp.dot((qa * dk).astype(jnp.bfloat16), bds,
                 preferred_element_type=f32)
    w1 = 1.0 / (1.0 + jnp.exp(-pd))
    o = va + w1 * dv
    o16 = o.astype(jnp.bfloat16)
    r = jnp.dot(o16, wo, preferred_element_type=f32)
    if bo.dtype == jnp.bfloat16:
        return (r + bo.astype(f32)).astype(jnp.bfloat16)
    return r + bo


def _attn_body(stg_ref, dyg_ref, te_ref,
               wqv1, wkv1, bq1, bv1, wo1, bo1,
               wqv2, wkv2, bq2, bv2, wo2, bo2,
               bds_ref, *rest):
    out_ref = rest[-1]
    stb = stg_ref[...]
    aux = stb[:, LA:LA + 1]
    ti = aux.astype(jnp.int32)
    oh = (ti == lax.broadcasted_iota(jnp.int32, (BA, T), 1))
    te16 = jnp.dot(oh.astype(jnp.bfloat16), te_ref[...],
                   preferred_element_type=jnp.float32
                   ).astype(jnp.bfloat16)
    a1 = dyg_ref[...].astype(jnp.bfloat16)
    a2 = stb.astype(jnp.bfloat16)
    bds = bds_ref[...]
    cond16 = _fuse(a1, te16, wqv1[...], wkv1[...], bq1[...], bv1[...],
                   bds, wo1[...], bo1[...])
    fused = _fuse(a2, cond16, wqv2[...], wkv2[...], bq2[...], bv2[...],
                  bds, wo2[...], bo2[...])
    out_ref[...] = jnp.where(aux >= 0.0, fused, jnp.float32(0.0))


def _attn_call(nblk, mout, stg, dyg, te16, w1, w2, bds16):
    full = lambda shape: pl.BlockSpec(shape, lambda i: tuple(0 for _ in shape))
    wspecs = lambda w: [full(x.shape) for x in w]
    specs = ([
        pl.BlockSpec((BA, DP), lambda i: (i, 0)),
        pl.BlockSpec((BA, DP), lambda i: (i, 0)),
        full((T, DP)),
    ] + wspecs(w1) + wspecs(w2) + [full((DP, DP))])
    args = [stg, dyg, te16] + list(w1) + list(w2) + [bds16]
    return pl.pallas_call(
        _attn_body,
        grid=(nblk,),
        in_specs=specs,
        out_specs=pl.BlockSpec((BA, D), lambda i: (i, 0)),
        out_shape=jax.ShapeDtypeStruct((mout, D), jnp.float32),
    )(*args)


def kernel(query_pts, query_times, buffer_voxel_index, static_features,
           dynamic_features, time_embeddings,
           f1_Wqkv, f1_bqkv, f1_Wo, f1_bo,
           f2_Wqkv, f2_bqkv, f2_Wo, f2_bo):
    m = query_pts.shape[0]
    mp = ((m + UNIT - 1) // UNIT) * UNIT
    nrow = mp // 128

    pts = query_pts.astype(jnp.float32)
    pidx = jnp.arange(mp - m, dtype=jnp.int32)
    padx = (pidx % 10).astype(jnp.float32) * 0.1 + 0.05
    pady = ((pidx // 10) % 10).astype(jnp.float32) * 0.1 + 0.05
    padz = ((pidx // 100) % 10).astype(jnp.float32) * 0.1 + 0.05
    qx2 = jnp.concatenate([pts[:, 0], padx]).reshape(nrow, 128)
    qy2 = jnp.concatenate([pts[:, 1], pady]).reshape(nrow, 128)
    qz2 = jnp.concatenate([pts[:, 2], padz]).reshape(nrow, 128)
    t2d = jnp.pad(query_times.astype(jnp.int32),
                  (0, mp - m)).reshape(nrow, 128)
    buf = buffer_voxel_index.astype(jnp.int32)
    stp = jnp.pad(static_features.astype(jnp.float32),
                  ((0, 0), (0, DP - D)))
    dyp = jnp.pad(dynamic_features.astype(jnp.float32),
                  ((0, 0), (0, DP - D)))

    h2d, tm2d = _hash_call(qx2, qy2, qz2, t2d, nrow)
    sv2d, aux2d = _lookup_call(h2d, buf, tm2d, nrow)

    ps = mp // PARTS
    pr = nrow // PARTS
    gathered = [_rows_call(sv2d[p * pr:(p + 1) * pr],
                           aux2d[p * pr:(p + 1) * pr], stp, dyp, ps)
                for p in range(PARTS)]

    bf16 = jnp.bfloat16

    def pad2(w, rows, cols):
        return jnp.pad(w, ((0, rows - w.shape[0]), (0, cols - w.shape[1])))

    def wpack(wqkv, bqkv, wo, bo, last):
        wq = pad2(wqkv[:, :D], DP, DP)
        wk = pad2(wqkv[:, D:2 * D], DP, DP)
        wv = pad2(wqkv[:, 2 * D:], DP, DP)
        wqv = jnp.concatenate([wq, wv], axis=1).astype(bf16)
        top = jnp.concatenate([-wk, -wv], axis=1)
        bot = jnp.concatenate([wk, wv], axis=1)
        wkv = jnp.concatenate([top, bot], axis=0).astype(bf16)
        bq = jnp.pad(bqkv[:D], (0, DP - D)).reshape(1, DP)
        bv = jnp.pad(bqkv[2 * D:], (0, DP - D)).reshape(1, DP)
        if last:
            wop = jnp.pad(wo, ((0, DP - D), (0, 0))).astype(bf16)
            bop = bo.reshape(1, D).astype(jnp.float32)
        else:
            wop = pad2(wo, DP, DP).astype(bf16)
            bop = jnp.pad(bo, (0, DP - D)).reshape(1, DP).astype(bf16)
        return (wqv, wkv, bq.astype(jnp.float32), bv.astype(jnp.float32),
                wop, bop)

    w1 = wpack(f1_Wqkv, f1_bqkv, f1_Wo, f1_bo, last=False)
    w2 = wpack(f2_Wqkv, f2_bqkv, f2_Wo, f2_bo, last=True)

    ri = jnp.arange(DP) // DH
    bd = jnp.where((ri[:, None] == ri[None, :])
                   & (jnp.arange(DP)[:, None] < D)
                   & (jnp.arange(DP)[None, :] < D),
                   1.0 / math.sqrt(DH), 0.0)
    bds16 = bd.astype(bf16)
    te16 = jnp.pad(time_embeddings, ((0, 0), (0, DP - D))).astype(bf16)

    nb = (m + BA - 1) // BA
    pb = ps // BA
    outs = []
    for p in range(PARTS):
        nbp = min(pb, nb - p * pb)
        if nbp <= 0:
            break
        stg_p, dyg_p = gathered[p]
        mout = min(ps, m - p * ps)
        outs.append(_attn_call(nbp, mout, stg_p, dyg_p, te16, w1, w2, bds16))
    if len(outs) == 1:
        return outs[0]
    return jnp.concatenate(outs, axis=0)


# speedup vs baseline: 1.0804x; 1.0195x over previous
import functools
import math

import jax
import jax.numpy as jnp
from jax import lax
from jax.experimental import pallas as pl
from jax.experimental.pallas import tpu as pltpu
from jax.experimental.pallas import tpu_sc as plsc

RES = 0.1
TABLE = 1 << 20
D = 120
DP = 128
LA = 120
T = 201
H = 8
DH = D // H
P0, P1, P2 = 73856093, 19349669, 83492791

NC, NS = 2, 16
NW = NC * NS
C = 128
GA = 8
BH = 4096
BA = 4096
PART_UNIT = NW * GA * C
PARTS = 4
UNIT = PARTS * PART_UNIT


def _hash_body(qx_ref, qy_ref, qz_ref, t_ref, h_ref, tm_ref):
    res = jnp.float32(RES)
    gx = jnp.floor(qx_ref[...] / res).astype(jnp.int32)
    gy = jnp.floor(qy_ref[...] / res).astype(jnp.int32)
    gz = jnp.floor(qz_ref[...] / res).astype(jnp.int32)
    s = gx * P0 + gy * P1 + gz * P2
    h_ref[...] = jnp.bitwise_and(s, TABLE - 1)
    tm_ref[...] = jnp.remainder(t_ref[...], T)


def _hash_call(qx2, qy2, qz2, t2d, nrow):
    rb = BH // 128
    grid = nrow // rb
    spec = pl.BlockSpec((rb, 128), lambda i: (i, 0))
    return pl.pallas_call(
        _hash_body,
        grid=(grid,),
        in_specs=[spec, spec, spec, spec],
        out_specs=[spec, spec],
        out_shape=(jax.ShapeDtypeStruct((nrow, 128), jnp.int32),
                   jax.ShapeDtypeStruct((nrow, 128), jnp.int32)),
    )(qx2, qy2, qz2, t2d)


def _pad_body(st_ref, dy_ref, stp_ref, dyp_ref):
    z = jnp.zeros((st_ref.shape[0], DP - D), jnp.float32)
    stp_ref[:, :D] = st_ref[...]
    stp_ref[:, D:] = z
    dyp_ref[:, :D] = dy_ref[...]
    dyp_ref[:, D:] = z


def _pad_call(st, dy):
    v = st.shape[0]
    rb = 1024
    grid = (v + rb - 1) // rb
    ispec = pl.BlockSpec((rb, D), lambda i: (i, 0))
    ospec = pl.BlockSpec((rb, DP), lambda i: (i, 0))
    return pl.pallas_call(
        _pad_body,
        grid=(grid,),
        in_specs=[ispec, ispec],
        out_specs=[ospec, ospec],
        out_shape=(jax.ShapeDtypeStruct((v, DP), jnp.float32),
                   jax.ShapeDtypeStruct((v, DP), jnp.float32)),
    )(st, dy)


def _aux_rows(vvr, tlr, svr, auxr):
    for i in range(GA):
        for j in range(128 // 16):
            sl = pl.ds(j * 16, 16)
            v = vvr[i, sl]
            t = tlr[i, sl]
            svr[i, sl] = jnp.maximum(v, 0)
            auxr[i, sl] = jnp.where(v >= 0, t, -1)


def _lookup_body(ngroup, h_hbm, buf_hbm, t_hbm, sv_hbm, aux_hbm,
                 h0, h1, t0, t1, vv0, vv1, sv0, sv1, ax0, ax1, sem0, sem1):
    cid = lax.axis_index("c")
    sid = lax.axis_index("s")
    wid = sid * NC + cid

    def pair(p, carry):
        r0 = (wid * ngroup + 2 * p) * GA
        r1 = r0 + GA
        pltpu.sync_copy(h_hbm.at[pl.ds(r0, GA)], h0)
        cps0 = [pltpu.async_copy(buf_hbm.at[h0.at[i]], vv0.at[i], sem0)
                for i in range(GA)]
        pltpu.sync_copy(h_hbm.at[pl.ds(r1, GA)], h1)
        cps1 = [pltpu.async_copy(buf_hbm.at[h1.at[i]], vv1.at[i], sem1)
                for i in range(GA)]
        pltpu.sync_copy(t_hbm.at[pl.ds(r0, GA)], t0)
        pltpu.sync_copy(t_hbm.at[pl.ds(r1, GA)], t1)
        for cp in cps0:
            cp.wait()
        _aux_rows(vv0, t0, sv0, ax0)
        pltpu.sync_copy(sv0, sv_hbm.at[pl.ds(r0, GA)])
        pltpu.sync_copy(ax0, aux_hbm.at[pl.ds(r0, GA)])
        for cp in cps1:
            cp.wait()
        _aux_rows(vv1, t1, sv1, ax1)
        pltpu.sync_copy(sv1, sv_hbm.at[pl.ds(r1, GA)])
        pltpu.sync_copy(ax1, aux_hbm.at[pl.ds(r1, GA)])
        return carry

    lax.fori_loop(0, ngroup // 2, pair, 0)


def _lookup_call(h2d, buf, t2d, nrow):
    ngroup = nrow // (NW * GA)
    mesh = plsc.VectorSubcoreMesh(core_axis_name="c", subcore_axis_name="s")
    i2d = lambda: pltpu.VMEM((GA, 128), jnp.int32)
    return pl.kernel(
        functools.partial(_lookup_body, ngroup),
        out_type=(
            jax.ShapeDtypeStruct((nrow, 128), jnp.int32),
            jax.ShapeDtypeStruct((nrow, 128), jnp.int32),
        ),
        mesh=mesh,
        compiler_params=pltpu.CompilerParams(use_tc_tiling_on_sc=True,
                                             needs_layout_passes=False),
        scratch_types=[i2d(), i2d(), i2d(), i2d(), i2d(), i2d(),
                       i2d(), i2d(), i2d(), i2d(),
                       pltpu.SemaphoreType.DMA, pltpu.SemaphoreType.DMA],
    )(h2d, buf, t2d)


def _embed_aux(axb, i, strow):
    cidx = jnp.full((16,), LA, jnp.int32)
    for j in range(C // 16):
        ridx = lax.broadcasted_iota(jnp.int32, (16,), 0) + j * 16
        af = axb[i, pl.ds(j * 16, 16)].astype(jnp.float32)
        plsc.store_scatter(strow, [ridx, cidx], af)


def _rows_body(nchunk, sv_hbm, aux_hbm, st_hbm, dy_hbm, stg_hbm, dyg_hbm,
               svb, axb, str0, str1, str2, dyr0, dyr1, dyr2,
               gs0, gs1, gs2, gd0, gd1, gd2, sem_w):
    cid = lax.axis_index("c")
    sid = lax.axis_index("s")
    wid = sid * NC + cid
    strs = (str0, str1, str2)
    dyrs = (dyr0, dyr1, dyr2)
    gss = (gs0, gs1, gs2)
    gds = (gd0, gd1, gd2)

    def octet(o, carry):
        base = wid * nchunk + o * 8
        pltpu.sync_copy(sv_hbm.at[pl.ds(base, 8)], svb)
        pltpu.sync_copy(aux_hbm.at[pl.ds(base, 8)], axb)

        def fire(i):
            k = i % 3
            return (pltpu.async_copy(st_hbm.at[svb.at[i]], strs[k], gss[k]),
                    pltpu.async_copy(dy_hbm.at[svb.at[i]], dyrs[k], gds[k]))

        cps = {i: fire(i) for i in range(3)}
        for i in range(8):
            k = i % 3
            cs, cd = cps[i]
            cs.wait()
            cd.wait()
            _embed_aux(axb, i, strs[k])
            ws = pltpu.async_copy(strs[k], stg_hbm.at[pl.ds((base + i) * C, C)],
                                  sem_w)
            wd = pltpu.async_copy(dyrs[k], dyg_hbm.at[pl.ds((base + i) * C, C)],
                                  sem_w)
            ws.wait()
            wd.wait()
            if i + 3 < 8:
                cps[i + 3] = fire(i + 3)
        return carry

    lax.fori_loop(0, nchunk // 8, octet, 0)


def _rows_call(sv2d, aux2d, stp, dyp, mp):
    nchunk = mp // (NW * C)
    mesh = plsc.VectorSubcoreMesh(core_axis_name="c", subcore_axis_name="s")
    rows = lambda: pltpu.VMEM((C, DP), jnp.float32)
    i2d = lambda: pltpu.VMEM((8, 128), jnp.int32)
    sem = pltpu.SemaphoreType.DMA
    return pl.kernel(
        functools.partial(_rows_body, nchunk),
        out_type=(
            jax.ShapeDtypeStruct((mp, DP), jnp.float32),
            jax.ShapeDtypeStruct((mp, DP), jnp.float32),
        ),
        mesh=mesh,
        compiler_params=pltpu.CompilerParams(use_tc_tiling_on_sc=True,
                                             needs_layout_passes=False),
        scratch_types=[i2d(), i2d(),
                       rows(), rows(), rows(), rows(), rows(), rows(),
                       sem, sem, sem, sem, sem, sem, sem],
    )(sv2d, aux2d, stp, dyp)


def _fuse(a16, b16, wqv, wkv, bq, bv, bds, wo, bo):
    f32 = jnp.float32
    qv = jnp.dot(a16, wqv, preferred_element_type=f32)
    qa = qv[:, :DP] + bq
    va = qv[:, DP:] + bv
    cat = jnp.concatenate([a16, b16], axis=1)
    kv = jnp.dot(cat, wkv, preferred_element_type=f32)
    dk = kv[:, :DP]
    dv = kv[:, DP:]
    pd = jnp.dot((qa * dk).astype(jnp.bfloat16), bds,
                 preferred_element_type=f32)
    w1 = 1.0 / (1.0 + jnp.exp(-pd))
    o = va + w1 * dv
    o16 = o.astype(jnp.bfloat16)
    r = jnp.dot(o16, wo, preferred_element_type=f32)
    if bo.dtype == jnp.bfloat16:
        return (r + bo.astype(f32)).astype(jnp.bfloat16)
    return r + bo


def _attn_body(stg_ref, dyg_ref, te_ref,
               wqv1, wkv1, bq1, bv1, wo1, bo1,
               wqv2, wkv2, bq2, bv2, wo2, bo2,
               bds_ref, *rest):
    out_ref = rest[-1]
    stb = stg_ref[...]
    aux = stb[:, LA:LA + 1]
    ti = aux.astype(jnp.int32)
    oh = (ti == lax.broadcasted_iota(jnp.int32, (BA, T), 1))
    te16 = jnp.dot(oh.astype(jnp.bfloat16), te_ref[...],
                   preferred_element_type=jnp.float32
                   ).astype(jnp.bfloat16)
    a1 = dyg_ref[...].astype(jnp.bfloat16)
    a2 = stb.astype(jnp.bfloat16)
    bds = bds_ref[...]
    cond16 = _fuse(a1, te16, wqv1[...], wkv1[...], bq1[...], bv1[...],
                   bds, wo1[...], bo1[...])
    fused = _fuse(a2, cond16, wqv2[...], wkv2[...], bq2[...], bv2[...],
                  bds, wo2[...], bo2[...])
    out_ref[...] = jnp.where(aux >= 0.0, fused, jnp.float32(0.0))


def _attn_call(nblk, mout, stg, dyg, te16, w1, w2, bds16):
    full = lambda shape: pl.BlockSpec(shape, lambda i: tuple(0 for _ in shape))
    wspecs = lambda w: [full(x.shape) for x in w]
    specs = ([
        pl.BlockSpec((BA, DP), lambda i: (i, 0)),
        pl.BlockSpec((BA, DP), lambda i: (i, 0)),
        full((T, DP)),
    ] + wspecs(w1) + wspecs(w2) + [full((DP, DP))])
    args = [stg, dyg, te16] + list(w1) + list(w2) + [bds16]
    return pl.pallas_call(
        _attn_body,
        grid=(nblk,),
        in_specs=specs,
        out_specs=pl.BlockSpec((BA, D), lambda i: (i, 0)),
        out_shape=jax.ShapeDtypeStruct((mout, D), jnp.float32),
    )(*args)


def kernel(query_pts, query_times, buffer_voxel_index, static_features,
           dynamic_features, time_embeddings,
           f1_Wqkv, f1_bqkv, f1_Wo, f1_bo,
           f2_Wqkv, f2_bqkv, f2_Wo, f2_bo):
    m = query_pts.shape[0]
    mp = ((m + UNIT - 1) // UNIT) * UNIT
    nrow = mp // 128

    pts = query_pts.astype(jnp.float32)
    pidx = jnp.arange(mp - m, dtype=jnp.int32)
    padx = (pidx % 10).astype(jnp.float32) * 0.1 + 0.05
    pady = ((pidx // 10) % 10).astype(jnp.float32) * 0.1 + 0.05
    padz = ((pidx // 100) % 10).astype(jnp.float32) * 0.1 + 0.05
    qx2 = jnp.concatenate([pts[:, 0], padx]).reshape(nrow, 128)
    qy2 = jnp.concatenate([pts[:, 1], pady]).reshape(nrow, 128)
    qz2 = jnp.concatenate([pts[:, 2], padz]).reshape(nrow, 128)
    t2d = jnp.pad(query_times.astype(jnp.int32),
                  (0, mp - m)).reshape(nrow, 128)
    buf = buffer_voxel_index.astype(jnp.int32)
    stp = jnp.pad(static_features.astype(jnp.float32),
                  ((0, 0), (0, DP - D)))
    dyp = jnp.pad(dynamic_features.astype(jnp.float32),
                  ((0, 0), (0, DP - D)))

    h2d, tm2d = _hash_call(qx2, qy2, qz2, t2d, nrow)
    sv2d, aux2d = _lookup_call(h2d, buf, tm2d, nrow)

    ps = mp // PARTS
    pr = nrow // PARTS
    gathered = [_rows_call(sv2d[p * pr:(p + 1) * pr],
                           aux2d[p * pr:(p + 1) * pr], stp, dyp, ps)
                for p in range(PARTS)]

    bf16 = jnp.bfloat16

    def pad2(w, rows, cols):
        return jnp.pad(w, ((0, rows - w.shape[0]), (0, cols - w.shape[1])))

    def wpack(wqkv, bqkv, wo, bo, last):
        wq = pad2(wqkv[:, :D], DP, DP)
        wk = pad2(wqkv[:, D:2 * D], DP, DP)
        wv = pad2(wqkv[:, 2 * D:], DP, DP)
        wqv = jnp.concatenate([wq, wv], axis=1).astype(bf16)
        top = jnp.concatenate([-wk, -wv], axis=1)
        bot = jnp.concatenate([wk, wv], axis=1)
        wkv = jnp.concatenate([top, bot], axis=0).astype(bf16)
        bq = jnp.pad(bqkv[:D], (0, DP - D)).reshape(1, DP)
        bv = jnp.pad(bqkv[2 * D:], (0, DP - D)).reshape(1, DP)
        if last:
            wop = jnp.pad(wo, ((0, DP - D), (0, 0))).astype(bf16)
            bop = bo.reshape(1, D).astype(jnp.float32)
        else:
            wop = pad2(wo, DP, DP).astype(bf16)
            bop = jnp.pad(bo, (0, DP - D)).reshape(1, DP).astype(bf16)
        return (wqv, wkv, bq.astype(jnp.float32), bv.astype(jnp.float32),
                wop, bop)

    w1 = wpack(f1_Wqkv, f1_bqkv, f1_Wo, f1_bo, last=False)
    w2 = wpack(f2_Wqkv, f2_bqkv, f2_Wo, f2_bo, last=True)

    ri = jnp.arange(DP) // DH
    bd = jnp.where((ri[:, None] == ri[None, :])
                   & (jnp.arange(DP)[:, None] < D)
                   & (jnp.arange(DP)[None, :] < D),
                   1.0 / math.sqrt(DH), 0.0)
    bds16 = bd.astype(bf16)
    te16 = jnp.pad(time_embeddings, ((0, 0), (0, DP - D))).astype(bf16)

    nb = (m + BA - 1) // BA
    pb = ps // BA
    outs = []
    for p in range(PARTS):
        nbp = min(pb, nb - p * pb)
        if nbp <= 0:
            break
        stg_p, dyg_p = gathered[p]
        mout = min(ps, m - p * ps)
        outs.append(_attn_call(nbp, mout, stg_p, dyg_p, te16, w1, w2, bds16))
    if len(outs) == 1:
        return outs[0]
    return jnp.concatenate(outs, axis=0)
